# baseline (speedup 1.0000x reference)
import functools

import jax
import jax.numpy as jnp
from jax import lax
from jax.experimental import pallas as pl
from jax.experimental.pallas import tpu as pltpu
from jax.experimental.pallas import tpu_sc as plsc

_NC = 2
_NS = 16
_NW = _NC * _NS
_LANES = 16
_KSC = 8192
_TILE = 16384


def _sc_tail_body(ctx, embed, vstart, rpt,
                  idx_hbm, emb_hbm, w_hbm, b_hbm, out_hbm,
                  idx_v, rows_v, w_v, b_v, o_v, sem, wsem):
    c = lax.axis_index("c")
    s = lax.axis_index("s")
    wid = s * _NC + c
    row0 = vstart + wid * rpt
    nch = embed // _LANES
    half = (ctx // 2 + 7) // 8 * 8

    pltpu.sync_copy(idx_hbm, idx_v)
    cp0 = pltpu.async_copy(emb_hbm.at[idx_v.at[pl.ds(0, half)]],
                           rows_v.at[pl.ds(0, half)], sem)
    cp1 = pltpu.async_copy(emb_hbm.at[idx_v.at[pl.ds(half, ctx - half)]],
                           rows_v.at[pl.ds(half, ctx - half)], sem)
    cpw = pltpu.async_copy(w_hbm.at[pl.ds(row0, rpt)], w_v, wsem)
    pltpu.sync_copy(b_hbm.at[pl.ds(row0, rpt)], b_v)
    cp0.wait()
    cp1.wait()

    def row_sum(j, acc):
        return tuple(acc[ch] + rows_v[j, pl.ds(ch * _LANES, _LANES)]
                     for ch in range(nch))

    e = lax.fori_loop(1, ctx, row_sum,
                      tuple(rows_v[0, pl.ds(ch * _LANES, _LANES)]
                            for ch in range(nch)))
    cpw.wait()

    lane = lax.iota(jnp.int32, _LANES)
    dnums = lax.GatherDimensionNumbers(
        offset_dims=(), collapsed_slice_dims=(0,), start_index_map=(0,))

    def hsum_splat(v):
        for k in (1, 2, 4, 8):
            perm = lax.gather(v, (lane ^ k).reshape(_LANES, 1), dnums, (1,),
                              mode=lax.GatherScatterMode.PROMISE_IN_BOUNDS)
            v = v + perm
        return v

    def group_body(g, carry):
        acc = jnp.zeros((_LANES,), jnp.float32)
        for j in range(_LANES):
            r = g * _LANES + j
            v = w_v[r, pl.ds(0, _LANES)] * e[0]
            for ch in range(1, nch):
                v = v + w_v[r, pl.ds(ch * _LANES, _LANES)] * e[ch]
            acc = jnp.where(lane == j, hsum_splat(v), acc)
        o_v[pl.ds(g * _LANES, _LANES)] = acc + b_v[pl.ds(g * _LANES, _LANES)]
        return carry

    lax.fori_loop(0, rpt // _LANES, group_body, 0)
    pltpu.sync_copy(o_v, out_hbm.at[pl.ds(wid * rpt, rpt)])


def _tail_sc(idx, emb, W, b):
    vocab, embed = W.shape
    ctx = idx.shape[0]
    rpt = _KSC // _NW
    vstart = vocab - _KSC
    mesh = plsc.VectorSubcoreMesh(
        core_axis_name="c", subcore_axis_name="s",
        num_cores=_NC, num_subcores=_NS)
    kern = pl.kernel(
        functools.partial(_sc_tail_body, ctx, embed, vstart, rpt),
        out_type=jax.ShapeDtypeStruct((_KSC,), jnp.float32),
        mesh=mesh,
        scratch_types=[
            pltpu.VMEM((ctx,), jnp.int32),
            pltpu.VMEM((ctx, embed), jnp.float32),
            pltpu.VMEM((rpt, embed), jnp.float32),
            pltpu.VMEM((rpt,), jnp.float32),
            pltpu.VMEM((rpt,), jnp.float32),
            pltpu.SemaphoreType.DMA,
            pltpu.SemaphoreType.DMA,
        ],
    )
    return kern(idx, emb, W, b)


def _head_body(ctx, idx_ref, emb_ref, w_ref, b_ref, o_ref, g_v, e_v, sem):
    @pl.when(pl.program_id(0) == 0)
    def _():
        cps = [pltpu.make_async_copy(emb_ref.at[pl.ds(idx_ref[i], 1)],
                                     g_v.at[pl.ds(i, 1)], sem)
               for i in range(ctx)]
        for cp in cps:
            cp.start()
        for cp in cps:
            cp.wait()
        e_v[...] = jnp.sum(g_v[...], axis=0, keepdims=True)

    o_ref[...] = jax.lax.dot_general(
        e_v[...], w_ref[...],
        dimension_numbers=(((1,), (1,)), ((), ())),
        preferred_element_type=jnp.float32) + b_ref[...].reshape(1, -1)


def _head_tc(idx, emb, W, b, width):
    vocab, embed = W.shape
    ctx = idx.shape[0]
    grid = (width + _TILE - 1) // _TILE
    return pl.pallas_call(
        functools.partial(_head_body, ctx),
        grid=(grid,),
        in_specs=[
            pl.BlockSpec(memory_space=pltpu.SMEM),
            pl.BlockSpec(memory_space=pl.ANY),
            pl.BlockSpec((_TILE, embed), lambda i: (i, 0)),
            pl.BlockSpec((_TILE,), lambda i: (i,)),
        ],
        out_specs=pl.BlockSpec((1, _TILE), lambda i: (0, i)),
        out_shape=jax.ShapeDtypeStruct((1, width), jnp.float32),
        scratch_shapes=[
            pltpu.VMEM((ctx, embed), jnp.float32),
            pltpu.VMEM((1, embed), jnp.float32),
            pltpu.SemaphoreType.DMA,
        ],
    )(idx, emb, W, b)


def kernel(inputs, emb, W, b):
    idx = inputs.astype(jnp.int32)
    out_tc = _head_tc(idx, emb, W, b, W.shape[0] - _KSC)
    out_sc = _tail_sc(idx, emb, W, b)
    return jnp.concatenate([out_tc, out_sc.reshape(1, -1)], axis=1)


# speedup vs baseline: 1.0542x; 1.0542x over previous
import functools

import jax
import jax.numpy as jnp
from jax import lax
from jax.experimental import pallas as pl
from jax.experimental.pallas import tpu as pltpu
from jax.experimental.pallas import tpu_sc as plsc

_LANES = 16
_SPLIT = 104


def _embed_sum_body(ctx, embed, idx_hbm, emb_hbm, out_hbm,
                    idx_v, rows_v, acc_v, sem):
    c = lax.axis_index("c")
    nch = embed // _LANES

    def gather_sum(offset, nrows):
        pltpu.sync_copy(idx_hbm.at[pl.ds(offset, nrows)],
                        idx_v.at[pl.ds(0, nrows)])
        pltpu.async_copy(emb_hbm.at[idx_v.at[pl.ds(0, nrows)]],
                         rows_v.at[pl.ds(0, nrows)], sem).wait()

        def row_sum(j, acc):
            return tuple(acc[ch] + rows_v[j, pl.ds(ch * _LANES, _LANES)]
                         for ch in range(nch))

        acc = lax.fori_loop(1, nrows, row_sum,
                            tuple(rows_v[0, pl.ds(ch * _LANES, _LANES)]
                                  for ch in range(nch)))
        for ch in range(nch):
            acc_v[pl.ds(ch * _LANES, _LANES)] = acc[ch]
        pltpu.sync_copy(acc_v, out_hbm.at[c])

    @pl.when(c == 0)
    def _():
        gather_sum(0, _SPLIT)

    @pl.when(c == 1)
    def _():
        gather_sum(_SPLIT, ctx - _SPLIT)


def _embed_sum_sc(idx, emb):
    embed = emb.shape[1]
    ctx = idx.shape[0]
    nmax = max(_SPLIT, ctx - _SPLIT)
    mesh = plsc.VectorSubcoreMesh(
        core_axis_name="c", subcore_axis_name="s",
        num_cores=2, num_subcores=1)
    kern = pl.kernel(
        functools.partial(_embed_sum_body, ctx, embed),
        out_type=jax.ShapeDtypeStruct((2, embed), jnp.float32),
        mesh=mesh,
        scratch_types=[
            pltpu.VMEM((nmax,), jnp.int32),
            pltpu.VMEM((nmax, embed), jnp.float32),
            pltpu.VMEM((embed,), jnp.float32),
            pltpu.SemaphoreType.DMA,
        ],
    )
    return kern(idx, emb)


_TILE = 16384


def _matvec_body(e_ref, w_ref, b_ref, o_ref):
    e = e_ref[0:1, :] + e_ref[1:2, :]
    o_ref[...] = jax.lax.dot_general(
        e, w_ref[...],
        dimension_numbers=(((1,), (1,)), ((), ())),
        preferred_element_type=jnp.float32) + b_ref[...].reshape(1, -1)


def _matvec_tc(partials, W, b):
    vocab, embed = W.shape
    grid = (vocab + _TILE - 1) // _TILE
    return pl.pallas_call(
        _matvec_body,
        grid=(grid,),
        in_specs=[
            pl.BlockSpec((2, embed), lambda i: (0, 0)),
            pl.BlockSpec((_TILE, embed), lambda i: (i, 0)),
            pl.BlockSpec((_TILE,), lambda i: (i,)),
        ],
        out_specs=pl.BlockSpec((1, _TILE), lambda i: (0, i)),
        out_shape=jax.ShapeDtypeStruct((1, vocab), jnp.float32),
    )(partials, W, b)


def kernel(inputs, emb, W, b):
    idx = inputs.astype(jnp.int32)
    partials = _embed_sum_sc(idx, emb)
    return _matvec_tc(partials, W, b)


# speedup vs baseline: 1.0873x; 1.0314x over previous
import functools

import jax
import jax.numpy as jnp
from jax import lax
from jax.experimental import pallas as pl
from jax.experimental.pallas import tpu as pltpu
from jax.experimental.pallas import tpu_sc as plsc

_LANES = 16


def _embed_sum_body(ctx, embed, idx_hbm, emb_hbm, out_hbm,
                    idx_v, rows_v, acc_v, sem0, sem1):
    nch = embed // _LANES
    half = (ctx // 2 + 7) // 8 * 8

    pltpu.sync_copy(idx_hbm, idx_v)
    cp0 = pltpu.async_copy(emb_hbm.at[idx_v.at[pl.ds(0, half)]],
                           rows_v.at[pl.ds(0, half)], sem0)
    cp1 = pltpu.async_copy(emb_hbm.at[idx_v.at[pl.ds(half, ctx - half)]],
                           rows_v.at[pl.ds(half, ctx - half)], sem1)

    def row_sum(j, acc):
        return tuple(acc[ch] + rows_v[j, pl.ds(ch * _LANES, _LANES)]
                     for ch in range(nch))

    cp0.wait()
    acc = lax.fori_loop(1, half, row_sum,
                        tuple(rows_v[0, pl.ds(ch * _LANES, _LANES)]
                              for ch in range(nch)))
    cp1.wait()
    acc = lax.fori_loop(half, ctx, row_sum, acc)
    for ch in range(nch):
        acc_v[pl.ds(ch * _LANES, _LANES)] = acc[ch]
    pltpu.sync_copy(acc_v, out_hbm.at[0])


def _embed_sum_sc(idx, emb):
    embed = emb.shape[1]
    ctx = idx.shape[0]
    mesh = plsc.VectorSubcoreMesh(
        core_axis_name="c", subcore_axis_name="s",
        num_cores=1, num_subcores=1)
    kern = pl.kernel(
        functools.partial(_embed_sum_body, ctx, embed),
        out_type=jax.ShapeDtypeStruct((1, embed), jnp.float32),
        mesh=mesh,
        scratch_types=[
            pltpu.VMEM((ctx,), jnp.int32),
            pltpu.VMEM((ctx, embed), jnp.float32),
            pltpu.VMEM((embed,), jnp.float32),
            pltpu.SemaphoreType.DMA,
            pltpu.SemaphoreType.DMA,
        ],
    )
    return kern(idx, emb)


_TILE = 16384


def _matvec_body(e_ref, w_ref, b_ref, o_ref):
    o_ref[...] = jax.lax.dot_general(
        e_ref[...], w_ref[...],
        dimension_numbers=(((1,), (1,)), ((), ())),
        preferred_element_type=jnp.float32) + b_ref[...].reshape(1, -1)


def _matvec_tc(partials, W, b):
    vocab, embed = W.shape
    grid = (vocab + _TILE - 1) // _TILE
    return pl.pallas_call(
        _matvec_body,
        grid=(grid,),
        in_specs=[
            pl.BlockSpec((1, embed), lambda i: (0, 0)),
            pl.BlockSpec((_TILE, embed), lambda i: (i, 0)),
            pl.BlockSpec((_TILE,), lambda i: (i,)),
        ],
        out_specs=pl.BlockSpec((1, _TILE), lambda i: (0, i)),
        out_shape=jax.ShapeDtypeStruct((1, vocab), jnp.float32),
    )(partials, W, b)


def kernel(inputs, emb, W, b):
    idx = inputs.astype(jnp.int32)
    partials = _embed_sum_sc(idx, emb)
    return _matvec_tc(partials, W, b)
